# baseline (speedup 1.0000x reference)
import functools

import jax
import jax.numpy as jnp
from jax import lax
from jax.experimental import pallas as pl
from jax.experimental.pallas import tpu as pltpu
from jax.experimental.pallas import tpu_sc as plsc

_V = 100000
_D = 32
_B = 16384
_W = 3125
_WIN = 3328
_WS = 3360
_BMAX = 96640
_TAIL0 = _V - 32
_PH = _B // 2
_IH = 4096
_PB = 512
_IMROWS = _PH + 8
_NSLOT = 48
_FLUSH_AT = 32
_PKDUMP = _PH


def _build():
    mesh = plsc.VectorSubcoreMesh(core_axis_name="c", subcore_axis_name="s")

    @functools.partial(
        pl.kernel,
        mesh=mesh,
        out_type=(
            jax.ShapeDtypeStruct((_D, _B), jnp.float32),
            jax.ShapeDtypeStruct((2 * _IMROWS, 128), jnp.float32),
        ),
        compiler_params=pltpu.CompilerParams(
            use_tc_tiling_on_sc=True, needs_layout_passes=False),
        scratch_types=[
            pltpu.VMEM((_D, _WS), jnp.float32),
            pltpu.VMEM((_IH,), jnp.int32),
            pltpu.VMEM((_D, 32), jnp.float32),
            pltpu.VMEM((_NSLOT,), jnp.int32),
            pltpu.VMEM((_NSLOT,), jnp.int32),
            pltpu.VMEM((_NSLOT, 128), jnp.float32),
            pltpu.VMEM((_D, 128), jnp.float32),
        ],
    )
    def gather_kernel(tbl_t, idx_hbm, tail_t, out_t, image, stage, idxb,
                      tailb, hits_pk, posb, records, outbuf):
        core = lax.axis_index("c")
        tile = lax.axis_index("s")
        lanes = lax.iota(jnp.int32, 16)
        im0 = core * _IMROWS

        def reset_hits():
            dumpv = jnp.full((16,), _PKDUMP, jnp.int32)
            for q in range(_NSLOT // 16):
                hits_pk[pl.ds(16 * q, 16)] = dumpv

        def flush():
            for g in range(_NSLOT // 16):
                pk = hits_pk[pl.ds(16 * g, 16)]
                cvec = pk // 16384
                posb[pl.ds(16 * g, 16)] = im0 + (pk & 16383)
                slot = lanes + (16 * g)
                for d in range(_D):
                    dvec = jnp.full((16,), d, jnp.int32)
                    vals = plsc.load_gather(stage, [dvec, cvec])
                    plsc.store_scatter(records, [slot, dvec], vals)
            pltpu.sync_copy(records, image.at[posb])
            reset_hits()

        pltpu.sync_copy(tail_t, tailb)
        reset_hits()

        for gen in range(2):
            r0 = (gen * 16 + tile) * _W
            base = jnp.minimum((r0 // 128) * 128, _BMAX)
            pltpu.sync_copy(tbl_t.at[:, pl.ds(base, _WIN)],
                            stage.at[:, pl.ds(0, _WIN)])
            if gen == 1:
                for d in range(_D):
                    for q in range(2):
                        stage[d, pl.ds(_WIN + 16 * q, 16)] = (
                            tailb[d, pl.ds(16 * q, 16)])

            for half in range(2):
                pltpu.sync_copy(
                    idx_hbm.at[pl.ds(core * _PH + half * _IH, _IH)], idxb)

                def scan_chunk(j, cnt, r0=r0, base=base, half=half):
                    iv = idxb[pl.ds(16 * j, 16)]
                    m = (iv >= r0) & (iv < r0 + _W)

                    @pl.when(jnp.any(m))
                    def _():
                        crel = jnp.where(iv >= _TAIL0,
                                         _WIN + iv - _TAIL0, iv - base)
                        pos = half * _IH + 16 * j + lanes
                        packed = jnp.where(m, crel * 16384 + pos, _PKDUMP)
                        key = 1 - m.astype(jnp.int32)
                        _, vs = plsc.sort_key_val(key, packed)
                        plsc.store_scatter(hits_pk, [cnt + lanes], vs)

                    cnt2 = cnt + plsc.all_reduce_population_count(m)
                    full = jnp.any(cnt2 >= _FLUSH_AT)

                    @pl.when(full)
                    def _():
                        flush()

                    return jnp.where(full, 0, cnt2)

                lax.fori_loop(0, _IH // 16, scan_chunk,
                              jnp.zeros((16,), jnp.int32))
                flush()

        plsc.subcore_barrier()

        def write_block(b, carry):
            row0 = tile * _PB + 128 * b
            for h in range(128 // _NSLOT + 1):
                n = min(_NSLOT, 128 - h * _NSLOT)
                pltpu.sync_copy(image.at[pl.ds(im0 + row0 + h * _NSLOT, n)],
                                records.at[pl.ds(0, n)])
                for g in range(-(-n // 16)):
                    pvec = lanes + (16 * g)
                    for d in range(_D):
                        dvec = jnp.full((16,), d, jnp.int32)
                        vals = plsc.load_gather(records, [pvec, dvec])
                        outbuf[d, pl.ds(h * _NSLOT + 16 * g, 16)] = vals
            pltpu.sync_copy(
                outbuf, out_t.at[:, pl.ds(core * _PH + row0, 128)])
            return carry

        lax.fori_loop(0, _PB // 128, write_block, jnp.int32(0))

    return gather_kernel


def kernel(theme_ids, table):
    gather_kernel = _build()
    out_t, _ = gather_kernel(table.T, theme_ids.astype(jnp.int32),
                             table[_TAIL0:, :].T)
    return out_t.T


# speedup vs baseline: 2.2506x; 2.2506x over previous
import functools

import jax
import jax.numpy as jnp
from jax import lax
from jax.experimental import pallas as pl
from jax.experimental.pallas import tpu as pltpu
from jax.experimental.pallas import tpu_sc as plsc

_V = 100000
_D = 32
_B = 16384
_W = 3125
_WIN = 3328
_WS = 3360
_BMAX = 96640
_TAIL0 = _V - 32
_PH = _B // 2
_IH = 4096
_PB = 512
_IMROWS = _PH + 8
_NSLOT = 48
_FLUSH_AT = 32
_PKDUMP = _PH


def _build():
    mesh = plsc.VectorSubcoreMesh(core_axis_name="c", subcore_axis_name="s")

    @functools.partial(
        pl.kernel,
        mesh=mesh,
        out_type=(
            jax.ShapeDtypeStruct((_D, _B), jnp.float32),
            jax.ShapeDtypeStruct((2 * _IMROWS, 128), jnp.float32),
        ),
        compiler_params=pltpu.CompilerParams(
            use_tc_tiling_on_sc=True, needs_layout_passes=False),
        scratch_types=[
            pltpu.VMEM((_D, _WS), jnp.float32),
            pltpu.VMEM((_IH,), jnp.int32),
            pltpu.VMEM((_D, 32), jnp.float32),
            pltpu.VMEM((_NSLOT,), jnp.int32),
            pltpu.VMEM((_NSLOT,), jnp.int32),
            pltpu.VMEM((_NSLOT, 128), jnp.float32),
            pltpu.VMEM((_D, 128), jnp.float32),
        ],
    )
    def gather_kernel(tbl_t, idx_hbm, tail_t, out_t, image, stage, idxb,
                      tailb, hits_pk, posb, records, outbuf):
        core = lax.axis_index("c")
        tile = lax.axis_index("s")
        lanes = lax.iota(jnp.int32, 16)
        im0 = core * _IMROWS

        def reset_hits():
            dumpv = jnp.full((16,), _PKDUMP, jnp.int32)
            for q in range(_NSLOT // 16):
                hits_pk[pl.ds(16 * q, 16)] = dumpv

        def flush():
            for g in range(_NSLOT // 16):
                pk = hits_pk[pl.ds(16 * g, 16)]
                cvec = pk // 16384
                posb[pl.ds(16 * g, 16)] = im0 + (pk & 16383)
                slot = lanes + (16 * g)
                for d in range(_D):
                    dvec = jnp.full((16,), d, jnp.int32)
                    vals = plsc.load_gather(stage, [dvec, cvec])
                    plsc.store_scatter(records, [slot, dvec], vals)
            reset_hits()

        pltpu.sync_copy(tail_t, tailb)
        reset_hits()

        for gen in range(2):
            r0 = (gen * 16 + tile) * _W
            base = jnp.minimum((r0 // 128) * 128, _BMAX)
            pltpu.sync_copy(tbl_t.at[:, pl.ds(base, _WIN)],
                            stage.at[:, pl.ds(0, _WIN)])
            if gen == 1:
                for d in range(_D):
                    for q in range(2):
                        stage[d, pl.ds(_WIN + 16 * q, 16)] = (
                            tailb[d, pl.ds(16 * q, 16)])

            for half in range(2):
                pltpu.sync_copy(
                    idx_hbm.at[pl.ds(core * _PH + half * _IH, _IH)], idxb)

                def scan_chunk(j, cnt, r0=r0, base=base, half=half):
                    iv = idxb[pl.ds(16 * j, 16)]
                    m = (iv >= r0) & (iv < r0 + _W)

                    @pl.when(jnp.any(m))
                    def _():
                        crel = jnp.where(iv >= _TAIL0,
                                         _WIN + iv - _TAIL0, iv - base)
                        pos = half * _IH + 16 * j + lanes
                        packed = jnp.where(m, crel * 16384 + pos, _PKDUMP)
                        key = 1 - m.astype(jnp.int32)
                        _, vs = plsc.sort_key_val(key, packed)
                        plsc.store_scatter(hits_pk, [cnt + lanes], vs)

                    cnt2 = cnt + plsc.all_reduce_population_count(m)
                    full = jnp.any(cnt2 >= _FLUSH_AT)

                    @pl.when(full)
                    def _():
                        flush()

                    return jnp.where(full, 0, cnt2)

                lax.fori_loop(0, _IH // 16, scan_chunk,
                              jnp.zeros((16,), jnp.int32))
                flush()

        plsc.subcore_barrier()

        def write_block(b, carry):
            row0 = tile * _PB + 128 * b
            for h in range(128 // _NSLOT + 1):
                n = min(_NSLOT, 128 - h * _NSLOT)
                pltpu.sync_copy(image.at[pl.ds(im0 + row0 + h * _NSLOT, n)],
                                records.at[pl.ds(0, n)])
                for g in range(-(-n // 16)):
                    pvec = lanes + (16 * g)
                    for d in range(_D):
                        dvec = jnp.full((16,), d, jnp.int32)
                        vals = plsc.load_gather(records, [pvec, dvec])
                        outbuf[d, pl.ds(h * _NSLOT + 16 * g, 16)] = vals
            pltpu.sync_copy(
                outbuf, out_t.at[:, pl.ds(core * _PH + row0, 128)])
            return carry

        lax.fori_loop(0, _PB // 128, write_block, jnp.int32(0))

    return gather_kernel


def kernel(theme_ids, table):
    gather_kernel = _build()
    out_t, _ = gather_kernel(table.T, theme_ids.astype(jnp.int32),
                             table[_TAIL0:, :].T)
    return out_t.T


# speedup vs baseline: 3.6352x; 1.6152x over previous
import functools

import jax
import jax.numpy as jnp
from jax import lax
from jax.experimental import pallas as pl
from jax.experimental.pallas import tpu as pltpu
from jax.experimental.pallas import tpu_sc as plsc

_NUM_THEMES = 100000
_EMBED_DIM = 32
_BATCH = 16384
_CHUNK = 128


def _build():
    info = plsc.get_sparse_core_info()
    nc, ns = info.num_cores, info.num_subcores
    nw = nc * ns
    b_per_w = _BATCH // nw
    nch = b_per_w // _CHUNK
    mesh = plsc.VectorSubcoreMesh(core_axis_name="c", subcore_axis_name="s")

    @functools.partial(
        pl.kernel,
        mesh=mesh,
        out_type=jax.ShapeDtypeStruct((_BATCH, _EMBED_DIM), jnp.float32),
        compiler_params=pltpu.CompilerParams(use_tc_tiling_on_sc=False),
        scratch_types=[
            pltpu.VMEM((b_per_w,), jnp.int32),
            pltpu.VMEM((b_per_w, _EMBED_DIM), jnp.float32),
            pltpu.SemaphoreType.DMA,
        ],
    )
    def gather_kernel(idx_hbm, table_hbm, out_hbm, idx_v, rows_v, sem):
        wid = lax.axis_index("s") * nc + lax.axis_index("c")
        base = wid * b_per_w
        pltpu.sync_copy(idx_hbm.at[pl.ds(base, b_per_w)], idx_v)
        copies = []
        for j in range(nch):
            copies.append(
                pltpu.async_copy(
                    table_hbm.at[idx_v.at[pl.ds(j * _CHUNK, _CHUNK)]],
                    rows_v.at[pl.ds(j * _CHUNK, _CHUNK)],
                    sem,
                )
            )
        for c in copies:
            c.wait()
        pltpu.sync_copy(rows_v, out_hbm.at[pl.ds(base, b_per_w)])

    return gather_kernel


def kernel(theme_ids, table):
    gather_kernel = _build()
    return gather_kernel(theme_ids.astype(jnp.int32), table)
